# baseline (speedup 1.0000x reference)
import functools

import jax
import jax.numpy as jnp
from jax import lax
from jax.experimental import pallas as pl
from jax.experimental.pallas import tpu as pltpu
from jax.experimental.pallas import tpu_sc as plsc

NC = 2
NS = 16
NW = NC * NS
CH = 80
NCHUNK = 5


def _sc_rows_gather(n_rows, d, n_ch_per_w):
  mesh = plsc.VectorSubcoreMesh(core_axis_name="c", subcore_axis_name="s")

  @functools.partial(
      pl.kernel,
      out_type=jax.ShapeDtypeStruct((n_rows, d), jnp.float32),
      mesh=mesh,
      scratch_types=[
          pltpu.VMEM((CH,), jnp.int32),
          pltpu.VMEM((CH, d), jnp.float32),
          pltpu.SemaphoreType.DMA,
      ],
  )
  def k(idx_h, tab_h, out_o, idx_v, row_v, sem):
    wid = lax.axis_index("s") * NC + lax.axis_index("c")

    def body(i, carry):
      b = (i * NW + wid) * CH
      pltpu.sync_copy(idx_h.at[pl.ds(b, CH)], idx_v)
      pltpu.async_copy(tab_h.at[idx_v], row_v, sem).wait()
      pltpu.sync_copy(row_v, out_o.at[pl.ds(b, CH)])
      return carry

    lax.fori_loop(0, n_ch_per_w, body, 0)

  return k


def _sc_edge_gather(per_w, n_ch, d):
  mesh = plsc.VectorSubcoreMesh(core_axis_name="c", subcore_axis_name="s")
  E = per_w * NW
  idx_t = pltpu.VMEM((CH,), jnp.int32)
  buf_t = pltpu.VMEM((CH, d), jnp.float32)

  sset = [idx_t, idx_t, idx_t, idx_t, buf_t, buf_t, buf_t, buf_t,
          pltpu.SemaphoreType.DMA, pltpu.SemaphoreType.DMA]

  @functools.partial(
      pl.kernel,
      out_type=(
          jax.ShapeDtypeStruct((E, d), jnp.float32),
          jax.ShapeDtypeStruct((E, d), jnp.float32),
          jax.ShapeDtypeStruct((E, d), jnp.float32),
          jax.ShapeDtypeStruct((E, d), jnp.float32),
      ),
      mesh=mesh,
      scratch_types=[list(sset), list(sset)],
  )
  def k(e0_h, e2_h, e4_h, e6_h, qa_h, rela_h, hid_h, tw1_h,
        qa_o, rel_o, hs_o, tw1_o, set_a, set_b):
    wid = lax.axis_index("s") * NC + lax.axis_index("c")
    base0 = wid * per_w
    tabs = (qa_h, rela_h, hid_h, tw1_h)
    outs = (qa_o, rel_o, hs_o, tw1_o)
    idx_hs = (e0_h, e2_h, e4_h, e6_h)

    def load_fire(b, S):
      for t in range(4):
        pltpu.sync_copy(idx_hs[t].at[pl.ds(b, CH)], S[t])
      for t in range(4):
        pltpu.async_copy(tabs[t].at[S[t]], S[4 + t], S[8])

    def waitg(S):
      for t in range(4):
        pltpu.make_async_copy(tabs[t].at[S[t]], S[4 + t], S[8]).wait()

    def fire_writes(b, S):
      for t in range(4):
        pltpu.async_copy(S[4 + t], outs[t].at[pl.ds(b, CH)], S[9])

    def waitw(b, S):
      for t in range(4):
        pltpu.make_async_copy(S[4 + t], outs[t].at[pl.ds(b, CH)], S[9]).wait()

    load_fire(base0, set_a)

    def body(j, carry):
      even = j % 2 == 0
      for S, T, mine in ((set_a, set_b, True), (set_b, set_a, False)):

        @pl.when(even == mine)
        def _(S=S, T=T):
          b = base0 + j * CH

          @pl.when(j + 1 < n_ch)
          def _():
            @pl.when(j >= 1)
            def _():
              waitw(base0 + (j - 1) * CH, T)

            load_fire(b + CH, T)

          waitg(S)
          fire_writes(b, S)

      return carry

    lax.fori_loop(0, n_ch, body, 0)
    last = n_ch - 1
    s_last = set_a if last % 2 == 0 else set_b
    s_prev = set_b if last % 2 == 0 else set_a
    waitw(base0 + (last - 1) * CH, s_prev)
    waitw(base0 + last * CH, s_last)

  return k


def _sc_scatter(per_w, n_ch, n_node, d, n_slices):
  mesh = plsc.VectorSubcoreMesh(core_axis_name="c", subcore_axis_name="s")
  n_rch = n_node // CH
  idx_t = pltpu.VMEM((CH,), jnp.int32)
  buf_t = pltpu.VMEM((CH, d), jnp.float32)

  @functools.partial(
      pl.kernel,
      out_type=jax.ShapeDtypeStruct((NC, n_node, d), jnp.float32),
      mesh=mesh,
      scratch_types=[
          [idx_t, buf_t, pltpu.SemaphoreType.DMA],
          [idx_t, buf_t, pltpu.SemaphoreType.DMA],
          pltpu.VMEM_SHARED((n_node, d), jnp.float32),
      ],
  )
  def k(*refs):
    obj_hs = refs[0:n_slices]
    msg_hs = refs[n_slices:2 * n_slices]
    zero_h = refs[2 * n_slices]
    agg_o = refs[2 * n_slices + 1]
    set_a, set_b, acc_sh = refs[2 * n_slices + 2:]
    c = lax.axis_index("c")
    s = lax.axis_index("s")
    wid = s * NC + c
    n_init = (n_rch + NS - 1) // NS

    def init_body(i, carry):
      cid = i * NS + s

      @pl.when(cid < n_rch)
      def _():
        pltpu.sync_copy(zero_h.at[pl.ds(cid * CH, CH)],
                        acc_sh.at[pl.ds(cid * CH, CH)])

      return carry

    lax.fori_loop(0, n_init, init_body, 0)
    plsc.subcore_barrier()
    base0 = wid * per_w

    for obj_h, msg_h in zip(obj_hs, msg_hs):
      def prefetch(b, S, obj_h=obj_h, msg_h=msg_h):
        pltpu.async_copy(obj_h.at[pl.ds(b, CH)], S[0], S[2])
        pltpu.async_copy(msg_h.at[pl.ds(b, CH)], S[1], S[2])

      def waitp(b, S, obj_h=obj_h, msg_h=msg_h):
        pltpu.make_async_copy(obj_h.at[pl.ds(b, CH)], S[0], S[2]).wait()
        pltpu.make_async_copy(msg_h.at[pl.ds(b, CH)], S[1], S[2]).wait()

      prefetch(base0, set_a)

      def body(j, carry):
        even = j % 2 == 0
        for S, T, mine in ((set_a, set_b, True), (set_b, set_a, False)):

          @pl.when(even == mine)
          def _(S=S, T=T):
            b = base0 + j * CH
            waitp(b, S)

            @pl.when(j + 1 < n_ch)
            def _():
              prefetch(b + CH, T)

            pltpu.sync_copy(S[1], acc_sh.at[S[0]], add=True)

        return carry

      lax.fori_loop(0, n_ch, body, 0)

    plsc.subcore_barrier()

    def drain_body(i, carry):
      cid = i * NS + s

      @pl.when(cid < n_rch)
      def _():
        pltpu.sync_copy(acc_sh.at[pl.ds(cid * CH, CH)],
                        agg_o.at[c, pl.ds(cid * CH, CH)])

      return carry

    lax.fori_loop(0, n_init, drain_body, 0)

  return k


def _matmul_block_kernel(x_r, w_r, out_r):
  r = jnp.dot(x_r[...], w_r[...], preferred_element_type=jnp.float32)
  out_r[...] = r.astype(out_r.dtype)


def _edge_block_kernel(rel_r, tw1_r, hs_r, qa_r, w1a_r, w2_r,
                       ws_r, wr_r, bias_r, out_r):
  bf16 = jnp.bfloat16
  f32 = jnp.float32
  rel = rel_r[...]
  tw1 = tw1_r[...]
  hs = hs_r[...]
  qa = qa_r[...]
  na = ws_r.shape[1]
  b1 = bias_r[0:1, :]
  b2 = bias_r[1:2, :]
  bqr = bias_r[2:3, 0:na]
  wa = bias_r[3:4, 0:na]
  wab = bias_r[4, 0]
  pre1 = (jnp.dot(rel.astype(bf16), w1a_r[...], preferred_element_type=f32)
          + tw1 + b1)
  t1 = jnp.where(pre1 >= 0, pre1, 0.01 * pre1)
  pre2 = jnp.dot(t1.astype(bf16), w2_r[...], preferred_element_type=f32) + b2
  h2 = jnp.where(pre2 >= 0, pre2, 0.01 * pre2)
  hr = h2 + rel
  att = (jnp.dot(hs.astype(bf16), ws_r[...], preferred_element_type=f32)
         + jnp.dot(hr.astype(bf16), wr_r[...], preferred_element_type=f32)
         + qa[:, 0:na] + bqr)
  att = jnp.maximum(att, 0.0)
  logit = jnp.sum(att * wa, axis=1, keepdims=True) + wab
  alpha = jax.nn.sigmoid(logit)
  out_r[...] = alpha * (hs + hr)


def _final_block_kernel(agg_r, agg2_r, wh_r, out_r):
  a = (agg_r[0] + agg_r[1]) + (agg2_r[0] + agg2_r[1])
  out_r[...] = jnp.dot(a, wh_r[...], preferred_element_type=jnp.float32)


def kernel(q_sub, q_rel, hidden, edges, n_node, rela_embed, time_embed,
           Ws, Wr, fuse_w1, fuse_b1, fuse_w2, fuse_b2, Wqr_w, Wqr_b,
           wa_w, wa_b, Wh):
  E = edges.shape[0]
  N = hidden.shape[0]
  D = hidden.shape[1]
  A = Ws.shape[0]
  NQ = q_rel.shape[0]

  e0 = edges[:, 0].astype(jnp.int32)
  e2 = edges[:, 2].astype(jnp.int32)
  e4 = edges[:, 4].astype(jnp.int32)
  e5 = edges[:, 5].astype(jnp.int32)
  e6 = edges[:, 6].astype(jnp.int32)

  blk = NW * CH
  NQP = ((NQ + blk - 1) // blk) * blk
  qrel_pad = jnp.zeros((NQP,), jnp.int32).at[:NQ].set(q_rel.astype(jnp.int32))
  re_q = _sc_rows_gather(NQP, D, NQP // blk)(qrel_pad, rela_embed)

  w1bT = fuse_w1[:, D:].T
  TW = time_embed.shape[0]
  tw1 = pl.pallas_call(
      _matmul_block_kernel,
      grid=(TW // 1000,),
      in_specs=[
          pl.BlockSpec((1000, time_embed.shape[1]), lambda i: (i, 0)),
          pl.BlockSpec((time_embed.shape[1], D), lambda i: (0, 0)),
      ],
      out_specs=pl.BlockSpec((1000, D), lambda i: (i, 0)),
      out_shape=jax.ShapeDtypeStruct((TW, D), jnp.float32),
  )(time_embed, w1bT)

  wqrT_pad = jnp.zeros((D, D), jnp.float32).at[:, :A].set(Wqr_w.T)
  qa_tab = pl.pallas_call(
      _matmul_block_kernel,
      grid=(NQP // 1024,),
      in_specs=[
          pl.BlockSpec((1024, D), lambda i: (i, 0)),
          pl.BlockSpec((D, D), lambda i: (0, 0)),
      ],
      out_specs=pl.BlockSpec((1024, D), lambda i: (i, 0)),
      out_shape=jax.ShapeDtypeStruct((NQP, D), jnp.float32),
  )(re_q, wqrT_pad)

  w1aT = fuse_w1[:, :D].T.astype(jnp.bfloat16)
  w2T = fuse_w2.T.astype(jnp.bfloat16)
  wsT = Ws.T.astype(jnp.bfloat16)
  wrT = Wr.T.astype(jnp.bfloat16)
  bias_pack = jnp.zeros((8, D), jnp.float32)
  bias_pack = bias_pack.at[0, :].set(fuse_b1)
  bias_pack = bias_pack.at[1, :].set(fuse_b2)
  bias_pack = bias_pack.at[2, :A].set(Wqr_b)
  bias_pack = bias_pack.at[3, :A].set(wa_w[0])
  bias_pack = bias_pack.at[4, 0].set(wa_b[0])

  ES = E // NCHUNK
  per_w = ES // NW
  n_ch = per_w // CH
  gather_fn = _sc_edge_gather(per_w, n_ch, D)
  BE = 512

  def edge_tc(rel_g, tw1_g, hs_g, qa_g):
    return pl.pallas_call(
        _edge_block_kernel,
        grid=(ES // BE,),
        in_specs=[
            pl.BlockSpec((BE, D), lambda i: (i, 0)),
            pl.BlockSpec((BE, D), lambda i: (i, 0)),
            pl.BlockSpec((BE, D), lambda i: (i, 0)),
            pl.BlockSpec((BE, D), lambda i: (i, 0)),
            pl.BlockSpec((D, D), lambda i: (0, 0)),
            pl.BlockSpec((D, D), lambda i: (0, 0)),
            pl.BlockSpec((D, A), lambda i: (0, 0)),
            pl.BlockSpec((D, A), lambda i: (0, 0)),
            pl.BlockSpec((8, D), lambda i: (0, 0)),
        ],
        out_specs=pl.BlockSpec((BE, D), lambda i: (i, 0)),
        out_shape=jax.ShapeDtypeStruct((ES, D), jnp.float32),
    )(rel_g, tw1_g, hs_g, qa_g, w1aT, w2T, wsT, wrT, bias_pack)

  msgs = []
  objs = []
  for k in range(NCHUNK):
    sl = slice(k * ES, (k + 1) * ES)
    qa_g, rel_g, hs_g, tw1_g = gather_fn(
        e0[sl], e2[sl], e4[sl], e6[sl], qa_tab, rela_embed, hidden, tw1)
    msgs.append(edge_tc(rel_g, tw1_g, hs_g, qa_g))
    objs.append(e5[sl])

  zeros_nd = jnp.zeros((N, D), jnp.float32)
  agg_a = _sc_scatter(per_w, n_ch, N, D, NCHUNK - 1)(
      *objs[:-1], *msgs[:-1], zeros_nd)
  agg_b = _sc_scatter(per_w, n_ch, N, D, 1)(objs[-1], msgs[-1], zeros_nd)

  RB = 1000
  out = pl.pallas_call(
      _final_block_kernel,
      grid=(N // RB,),
      in_specs=[
          pl.BlockSpec((NC, RB, D), lambda i: (0, i, 0)),
          pl.BlockSpec((NC, RB, D), lambda i: (0, i, 0)),
          pl.BlockSpec((D, D), lambda i: (0, 0)),
      ],
      out_specs=pl.BlockSpec((RB, D), lambda i: (i, 0)),
      out_shape=jax.ShapeDtypeStruct((N, D), jnp.float32),
  )(agg_a, agg_b, Wh.T)
  return out


# speedup vs baseline: 1.0343x; 1.0343x over previous
import functools

import jax
import jax.numpy as jnp
from jax import lax
from jax.experimental import pallas as pl
from jax.experimental.pallas import tpu as pltpu
from jax.experimental.pallas import tpu_sc as plsc

NC = 2
NS = 16
NW = NC * NS
CH = 80
NCHUNK = 5


def _sc_rows_gather(n_rows, d, n_ch_per_w):
  mesh = plsc.VectorSubcoreMesh(core_axis_name="c", subcore_axis_name="s")

  @functools.partial(
      pl.kernel,
      out_type=jax.ShapeDtypeStruct((n_rows, d), jnp.float32),
      mesh=mesh,
      scratch_types=[
          pltpu.VMEM((CH,), jnp.int32),
          pltpu.VMEM((CH, d), jnp.float32),
          pltpu.SemaphoreType.DMA,
      ],
  )
  def k(idx_h, tab_h, out_o, idx_v, row_v, sem):
    wid = lax.axis_index("s") * NC + lax.axis_index("c")

    def body(i, carry):
      b = (i * NW + wid) * CH
      pltpu.sync_copy(idx_h.at[pl.ds(b, CH)], idx_v)
      pltpu.async_copy(tab_h.at[idx_v], row_v, sem).wait()
      pltpu.sync_copy(row_v, out_o.at[pl.ds(b, CH)])
      return carry

    lax.fori_loop(0, n_ch_per_w, body, 0)

  return k


def _sc_edge_gather(per_w, n_ch, d):
  mesh = plsc.VectorSubcoreMesh(core_axis_name="c", subcore_axis_name="s")
  E = per_w * NW
  idx_t = pltpu.VMEM((CH,), jnp.int32)
  buf_t = pltpu.VMEM((CH, d), jnp.float32)

  sset = [idx_t, idx_t, idx_t, idx_t, buf_t, buf_t, buf_t, buf_t,
          pltpu.SemaphoreType.DMA, pltpu.SemaphoreType.DMA]

  @functools.partial(
      pl.kernel,
      out_type=(
          jax.ShapeDtypeStruct((E, d), jnp.float32),
          jax.ShapeDtypeStruct((E, d), jnp.float32),
          jax.ShapeDtypeStruct((E, d), jnp.float32),
          jax.ShapeDtypeStruct((E, d), jnp.float32),
      ),
      mesh=mesh,
      scratch_types=[list(sset), list(sset)],
  )
  def k(e0_h, e2_h, e4_h, e6_h, qa_h, rela_h, hid_h, tw1_h,
        qa_o, rel_o, hs_o, tw1_o, set_a, set_b):
    wid = lax.axis_index("s") * NC + lax.axis_index("c")
    base0 = wid * per_w
    tabs = (qa_h, rela_h, hid_h, tw1_h)
    outs = (qa_o, rel_o, hs_o, tw1_o)
    idx_hs = (e0_h, e2_h, e4_h, e6_h)

    def load_fire(b, S):
      for t in range(4):
        pltpu.sync_copy(idx_hs[t].at[pl.ds(b, CH)], S[t])
      for t in range(4):
        pltpu.async_copy(tabs[t].at[S[t]], S[4 + t], S[8])

    def waitg(S):
      for t in range(4):
        pltpu.make_async_copy(tabs[t].at[S[t]], S[4 + t], S[8]).wait()

    def fire_writes(b, S):
      for t in range(4):
        pltpu.async_copy(S[4 + t], outs[t].at[pl.ds(b, CH)], S[9])

    def waitw(b, S):
      for t in range(4):
        pltpu.make_async_copy(S[4 + t], outs[t].at[pl.ds(b, CH)], S[9]).wait()

    load_fire(base0, set_a)

    def body(j, carry):
      even = j % 2 == 0
      for S, T, mine in ((set_a, set_b, True), (set_b, set_a, False)):

        @pl.when(even == mine)
        def _(S=S, T=T):
          b = base0 + j * CH
          waitg(S)

          @pl.when(j + 1 < n_ch)
          def _():
            @pl.when(j >= 1)
            def _():
              waitw(base0 + (j - 1) * CH, T)

            load_fire(b + CH, T)

          fire_writes(b, S)

      return carry

    lax.fori_loop(0, n_ch, body, 0)
    last = n_ch - 1
    s_last = set_a if last % 2 == 0 else set_b
    s_prev = set_b if last % 2 == 0 else set_a
    waitw(base0 + (last - 1) * CH, s_prev)
    waitw(base0 + last * CH, s_last)

  return k


def _sc_scatter(per_w, n_ch, n_node, d, n_slices):
  mesh = plsc.VectorSubcoreMesh(core_axis_name="c", subcore_axis_name="s")
  n_rch = n_node // CH
  idx_t = pltpu.VMEM((CH,), jnp.int32)
  buf_t = pltpu.VMEM((CH, d), jnp.float32)

  @functools.partial(
      pl.kernel,
      out_type=jax.ShapeDtypeStruct((NC, n_node, d), jnp.float32),
      mesh=mesh,
      scratch_types=[
          [idx_t, buf_t, pltpu.SemaphoreType.DMA],
          [idx_t, buf_t, pltpu.SemaphoreType.DMA],
          pltpu.VMEM_SHARED((n_node, d), jnp.float32),
      ],
  )
  def k(*refs):
    obj_hs = refs[0:n_slices]
    msg_hs = refs[n_slices:2 * n_slices]
    zero_h = refs[2 * n_slices]
    agg_o = refs[2 * n_slices + 1]
    set_a, set_b, acc_sh = refs[2 * n_slices + 2:]
    c = lax.axis_index("c")
    s = lax.axis_index("s")
    wid = s * NC + c
    n_init = (n_rch + NS - 1) // NS

    def init_body(i, carry):
      cid = i * NS + s

      @pl.when(cid < n_rch)
      def _():
        pltpu.sync_copy(zero_h.at[pl.ds(cid * CH, CH)],
                        acc_sh.at[pl.ds(cid * CH, CH)])

      return carry

    lax.fori_loop(0, n_init, init_body, 0)
    plsc.subcore_barrier()
    base0 = wid * per_w

    for obj_h, msg_h in zip(obj_hs, msg_hs):
      def prefetch(b, S, obj_h=obj_h, msg_h=msg_h):
        pltpu.async_copy(obj_h.at[pl.ds(b, CH)], S[0], S[2])
        pltpu.async_copy(msg_h.at[pl.ds(b, CH)], S[1], S[2])

      def waitp(b, S, obj_h=obj_h, msg_h=msg_h):
        pltpu.make_async_copy(obj_h.at[pl.ds(b, CH)], S[0], S[2]).wait()
        pltpu.make_async_copy(msg_h.at[pl.ds(b, CH)], S[1], S[2]).wait()

      prefetch(base0, set_a)

      def body(j, carry):
        even = j % 2 == 0
        for S, T, mine in ((set_a, set_b, True), (set_b, set_a, False)):

          @pl.when(even == mine)
          def _(S=S, T=T):
            b = base0 + j * CH
            waitp(b, S)

            @pl.when(j + 1 < n_ch)
            def _():
              prefetch(b + CH, T)

            pltpu.sync_copy(S[1], acc_sh.at[S[0]], add=True)

        return carry

      lax.fori_loop(0, n_ch, body, 0)

    plsc.subcore_barrier()

    def drain_body(i, carry):
      cid = i * NS + s

      @pl.when(cid < n_rch)
      def _():
        pltpu.sync_copy(acc_sh.at[pl.ds(cid * CH, CH)],
                        agg_o.at[c, pl.ds(cid * CH, CH)])

      return carry

    lax.fori_loop(0, n_init, drain_body, 0)

  return k


def _matmul_block_kernel(x_r, w_r, out_r):
  r = jnp.dot(x_r[...], w_r[...], preferred_element_type=jnp.float32)
  out_r[...] = r.astype(out_r.dtype)


def _edge_block_kernel(rel_r, tw1_r, hs_r, qa_r, w1a_r, w2_r,
                       ws_r, wr_r, bias_r, out_r):
  bf16 = jnp.bfloat16
  f32 = jnp.float32
  rel = rel_r[...]
  tw1 = tw1_r[...]
  hs = hs_r[...]
  qa = qa_r[...]
  na = ws_r.shape[1]
  b1 = bias_r[0:1, :]
  b2 = bias_r[1:2, :]
  bqr = bias_r[2:3, 0:na]
  wa = bias_r[3:4, 0:na]
  wab = bias_r[4, 0]
  pre1 = (jnp.dot(rel.astype(bf16), w1a_r[...], preferred_element_type=f32)
          + tw1 + b1)
  t1 = jnp.where(pre1 >= 0, pre1, 0.01 * pre1)
  pre2 = jnp.dot(t1.astype(bf16), w2_r[...], preferred_element_type=f32) + b2
  h2 = jnp.where(pre2 >= 0, pre2, 0.01 * pre2)
  hr = h2 + rel
  att = (jnp.dot(hs.astype(bf16), ws_r[...], preferred_element_type=f32)
         + jnp.dot(hr.astype(bf16), wr_r[...], preferred_element_type=f32)
         + qa[:, 0:na] + bqr)
  att = jnp.maximum(att, 0.0)
  logit = jnp.sum(att * wa, axis=1, keepdims=True) + wab
  alpha = jax.nn.sigmoid(logit)
  out_r[...] = alpha * (hs + hr)


def _final_block_kernel(agg_r, agg2_r, wh_r, out_r):
  a = (agg_r[0] + agg_r[1]) + (agg2_r[0] + agg2_r[1])
  out_r[...] = jnp.dot(a, wh_r[...], preferred_element_type=jnp.float32)


def kernel(q_sub, q_rel, hidden, edges, n_node, rela_embed, time_embed,
           Ws, Wr, fuse_w1, fuse_b1, fuse_w2, fuse_b2, Wqr_w, Wqr_b,
           wa_w, wa_b, Wh):
  E = edges.shape[0]
  N = hidden.shape[0]
  D = hidden.shape[1]
  A = Ws.shape[0]
  NQ = q_rel.shape[0]

  e0 = edges[:, 0].astype(jnp.int32)
  e2 = edges[:, 2].astype(jnp.int32)
  e4 = edges[:, 4].astype(jnp.int32)
  e5 = edges[:, 5].astype(jnp.int32)
  e6 = edges[:, 6].astype(jnp.int32)

  blk = NW * CH
  NQP = ((NQ + blk - 1) // blk) * blk
  qrel_pad = jnp.zeros((NQP,), jnp.int32).at[:NQ].set(q_rel.astype(jnp.int32))
  re_q = _sc_rows_gather(NQP, D, NQP // blk)(qrel_pad, rela_embed)

  w1bT = fuse_w1[:, D:].T
  TW = time_embed.shape[0]
  tw1 = pl.pallas_call(
      _matmul_block_kernel,
      grid=(TW // 1000,),
      in_specs=[
          pl.BlockSpec((1000, time_embed.shape[1]), lambda i: (i, 0)),
          pl.BlockSpec((time_embed.shape[1], D), lambda i: (0, 0)),
      ],
      out_specs=pl.BlockSpec((1000, D), lambda i: (i, 0)),
      out_shape=jax.ShapeDtypeStruct((TW, D), jnp.float32),
  )(time_embed, w1bT)

  wqrT_pad = jnp.zeros((D, D), jnp.float32).at[:, :A].set(Wqr_w.T)
  qa_tab = pl.pallas_call(
      _matmul_block_kernel,
      grid=(NQP // 1024,),
      in_specs=[
          pl.BlockSpec((1024, D), lambda i: (i, 0)),
          pl.BlockSpec((D, D), lambda i: (0, 0)),
      ],
      out_specs=pl.BlockSpec((1024, D), lambda i: (i, 0)),
      out_shape=jax.ShapeDtypeStruct((NQP, D), jnp.float32),
  )(re_q, wqrT_pad)

  w1aT = fuse_w1[:, :D].T.astype(jnp.bfloat16)
  w2T = fuse_w2.T.astype(jnp.bfloat16)
  wsT = Ws.T.astype(jnp.bfloat16)
  wrT = Wr.T.astype(jnp.bfloat16)
  bias_pack = jnp.zeros((8, D), jnp.float32)
  bias_pack = bias_pack.at[0, :].set(fuse_b1)
  bias_pack = bias_pack.at[1, :].set(fuse_b2)
  bias_pack = bias_pack.at[2, :A].set(Wqr_b)
  bias_pack = bias_pack.at[3, :A].set(wa_w[0])
  bias_pack = bias_pack.at[4, 0].set(wa_b[0])

  ES = E // NCHUNK
  per_w = ES // NW
  n_ch = per_w // CH
  gather_fn = _sc_edge_gather(per_w, n_ch, D)
  BE = 512

  def edge_tc(rel_g, tw1_g, hs_g, qa_g):
    return pl.pallas_call(
        _edge_block_kernel,
        grid=(ES // BE,),
        in_specs=[
            pl.BlockSpec((BE, D), lambda i: (i, 0)),
            pl.BlockSpec((BE, D), lambda i: (i, 0)),
            pl.BlockSpec((BE, D), lambda i: (i, 0)),
            pl.BlockSpec((BE, D), lambda i: (i, 0)),
            pl.BlockSpec((D, D), lambda i: (0, 0)),
            pl.BlockSpec((D, D), lambda i: (0, 0)),
            pl.BlockSpec((D, A), lambda i: (0, 0)),
            pl.BlockSpec((D, A), lambda i: (0, 0)),
            pl.BlockSpec((8, D), lambda i: (0, 0)),
        ],
        out_specs=pl.BlockSpec((BE, D), lambda i: (i, 0)),
        out_shape=jax.ShapeDtypeStruct((ES, D), jnp.float32),
    )(rel_g, tw1_g, hs_g, qa_g, w1aT, w2T, wsT, wrT, bias_pack)

  msgs = []
  objs = []
  for k in range(NCHUNK):
    sl = slice(k * ES, (k + 1) * ES)
    qa_g, rel_g, hs_g, tw1_g = gather_fn(
        e0[sl], e2[sl], e4[sl], e6[sl], qa_tab, rela_embed, hidden, tw1)
    msgs.append(edge_tc(rel_g, tw1_g, hs_g, qa_g))
    objs.append(e5[sl])

  zeros_nd = jnp.zeros((N, D), jnp.float32)
  agg_a = _sc_scatter(per_w, n_ch, N, D, NCHUNK - 1)(
      *objs[:-1], *msgs[:-1], zeros_nd)
  agg_b = _sc_scatter(per_w, n_ch, N, D, 1)(objs[-1], msgs[-1], zeros_nd)

  RB = 1000
  out = pl.pallas_call(
      _final_block_kernel,
      grid=(N // RB,),
      in_specs=[
          pl.BlockSpec((NC, RB, D), lambda i: (0, i, 0)),
          pl.BlockSpec((NC, RB, D), lambda i: (0, i, 0)),
          pl.BlockSpec((D, D), lambda i: (0, 0)),
      ],
      out_specs=pl.BlockSpec((RB, D), lambda i: (i, 0)),
      out_shape=jax.ShapeDtypeStruct((N, D), jnp.float32),
  )(agg_a, agg_b, Wh.T)
  return out


# speedup vs baseline: 1.0381x; 1.0036x over previous
import functools

import jax
import jax.numpy as jnp
from jax import lax
from jax.experimental import pallas as pl
from jax.experimental.pallas import tpu as pltpu
from jax.experimental.pallas import tpu_sc as plsc

NC = 2
NS = 16
NW = NC * NS
CH = 80
NCHUNK = 5


def _sc_rows_gather(n_rows, d, n_ch_per_w):
  mesh = plsc.VectorSubcoreMesh(core_axis_name="c", subcore_axis_name="s")

  @functools.partial(
      pl.kernel,
      out_type=jax.ShapeDtypeStruct((n_rows, d), jnp.float32),
      mesh=mesh,
      scratch_types=[
          pltpu.VMEM((CH,), jnp.int32),
          pltpu.VMEM((CH, d), jnp.float32),
          pltpu.SemaphoreType.DMA,
      ],
  )
  def k(idx_h, tab_h, out_o, idx_v, row_v, sem):
    wid = lax.axis_index("s") * NC + lax.axis_index("c")

    def body(i, carry):
      b = (i * NW + wid) * CH
      pltpu.sync_copy(idx_h.at[pl.ds(b, CH)], idx_v)
      pltpu.async_copy(tab_h.at[idx_v], row_v, sem).wait()
      pltpu.sync_copy(row_v, out_o.at[pl.ds(b, CH)])
      return carry

    lax.fori_loop(0, n_ch_per_w, body, 0)

  return k


def _sc_edge_gather(per_w, n_ch, d):
  mesh = plsc.VectorSubcoreMesh(core_axis_name="c", subcore_axis_name="s")
  E = per_w * NW
  idx_t = pltpu.VMEM((CH,), jnp.int32)
  buf_t = pltpu.VMEM((CH, d), jnp.float32)

  sset = [idx_t, idx_t, idx_t, idx_t, buf_t, buf_t, buf_t, buf_t,
          pltpu.SemaphoreType.DMA, pltpu.SemaphoreType.DMA]

  @functools.partial(
      pl.kernel,
      out_type=(
          jax.ShapeDtypeStruct((E, d), jnp.float32),
          jax.ShapeDtypeStruct((E, d), jnp.float32),
          jax.ShapeDtypeStruct((E, d), jnp.float32),
          jax.ShapeDtypeStruct((E, d), jnp.float32),
      ),
      mesh=mesh,
      scratch_types=[list(sset), list(sset)],
  )
  def k(e0_h, e2_h, e4_h, e6_h, qa_h, rela_h, hid_h, tw1_h,
        qa_o, rel_o, hs_o, tw1_o, set_a, set_b):
    wid = lax.axis_index("s") * NC + lax.axis_index("c")
    base0 = wid * per_w
    tabs = (qa_h, rela_h, hid_h, tw1_h)
    outs = (qa_o, rel_o, hs_o, tw1_o)
    idx_hs = (e0_h, e2_h, e4_h, e6_h)

    def load_fire(b, S):
      for t in range(4):
        pltpu.sync_copy(idx_hs[t].at[pl.ds(b, CH)], S[t])
      for t in range(4):
        pltpu.async_copy(tabs[t].at[S[t]], S[4 + t], S[8])

    def waitg(S):
      for t in range(4):
        pltpu.make_async_copy(tabs[t].at[S[t]], S[4 + t], S[8]).wait()

    def fire_writes(b, S):
      for t in range(4):
        pltpu.async_copy(S[4 + t], outs[t].at[pl.ds(b, CH)], S[9])

    def waitw(b, S):
      for t in range(4):
        pltpu.make_async_copy(S[4 + t], outs[t].at[pl.ds(b, CH)], S[9]).wait()

    load_fire(base0, set_a)

    def body(j, carry):
      even = j % 2 == 0
      for S, T, mine in ((set_a, set_b, True), (set_b, set_a, False)):

        @pl.when(even == mine)
        def _(S=S, T=T):
          b = base0 + j * CH
          waitg(S)

          @pl.when(j + 1 < n_ch)
          def _():
            @pl.when(j >= 1)
            def _():
              waitw(base0 + (j - 1) * CH, T)

            load_fire(b + CH, T)

          fire_writes(b, S)

      return carry

    lax.fori_loop(0, n_ch, body, 0)
    last = n_ch - 1
    s_last = set_a if last % 2 == 0 else set_b
    s_prev = set_b if last % 2 == 0 else set_a
    waitw(base0 + (last - 1) * CH, s_prev)
    waitw(base0 + last * CH, s_last)

  return k


def _sc_scatter(per_w, n_ch, n_node, d, n_slices):
  mesh = plsc.VectorSubcoreMesh(core_axis_name="c", subcore_axis_name="s")
  n_rch = n_node // CH
  idx_t = pltpu.VMEM((CH,), jnp.int32)
  buf_t = pltpu.VMEM((CH, d), jnp.float32)

  @functools.partial(
      pl.kernel,
      out_type=jax.ShapeDtypeStruct((NC, n_node, d), jnp.float32),
      mesh=mesh,
      scratch_types=[
          [idx_t, buf_t, pltpu.SemaphoreType.DMA],
          [idx_t, buf_t, pltpu.SemaphoreType.DMA],
          pltpu.VMEM_SHARED((n_node, d), jnp.float32),
      ],
  )
  def k(*refs):
    obj_hs = refs[0:n_slices]
    msg_hs = refs[n_slices:2 * n_slices]
    zero_h = refs[2 * n_slices]
    agg_o = refs[2 * n_slices + 1]
    set_a, set_b, acc_sh = refs[2 * n_slices + 2:]
    c = lax.axis_index("c")
    s = lax.axis_index("s")
    wid = s * NC + c
    n_init = (n_rch + NS - 1) // NS

    def init_body(i, carry):
      cid = i * NS + s

      @pl.when(cid < n_rch)
      def _():
        pltpu.sync_copy(zero_h.at[pl.ds(cid * CH, CH)],
                        acc_sh.at[pl.ds(cid * CH, CH)])

      return carry

    lax.fori_loop(0, n_init, init_body, 0)
    plsc.subcore_barrier()
    base0 = wid * per_w

    for obj_h, msg_h in zip(obj_hs, msg_hs):
      def prefetch(b, S, obj_h=obj_h, msg_h=msg_h):
        pltpu.async_copy(obj_h.at[pl.ds(b, CH)], S[0], S[2])
        pltpu.async_copy(msg_h.at[pl.ds(b, CH)], S[1], S[2])

      def waitp(b, S, obj_h=obj_h, msg_h=msg_h):
        pltpu.make_async_copy(obj_h.at[pl.ds(b, CH)], S[0], S[2]).wait()
        pltpu.make_async_copy(msg_h.at[pl.ds(b, CH)], S[1], S[2]).wait()

      prefetch(base0, set_a)

      def body(j, carry):
        even = j % 2 == 0
        for S, T, mine in ((set_a, set_b, True), (set_b, set_a, False)):

          @pl.when(even == mine)
          def _(S=S, T=T):
            b = base0 + j * CH
            waitp(b, S)

            @pl.when(j + 1 < n_ch)
            def _():
              prefetch(b + CH, T)

            pltpu.sync_copy(S[1], acc_sh.at[S[0]], add=True)

        return carry

      lax.fori_loop(0, n_ch, body, 0)

    plsc.subcore_barrier()

    def drain_body(i, carry):
      cid = i * NS + s

      @pl.when(cid < n_rch)
      def _():
        pltpu.sync_copy(acc_sh.at[pl.ds(cid * CH, CH)],
                        agg_o.at[c, pl.ds(cid * CH, CH)])

      return carry

    lax.fori_loop(0, n_init, drain_body, 0)

  return k


def _matmul_block_kernel(x_r, w_r, out_r):
  r = jnp.dot(x_r[...], w_r[...], preferred_element_type=jnp.float32)
  out_r[...] = r.astype(out_r.dtype)


def _edge_block_kernel(rel_r, tw1_r, hs_r, qa_r, w1a_r, w2_r,
                       ws_r, wr_r, bias_r, out_r):
  bf16 = jnp.bfloat16
  f32 = jnp.float32
  rel = rel_r[...]
  tw1 = tw1_r[...]
  hs = hs_r[...]
  qa = qa_r[...]
  na = ws_r.shape[1]
  b1 = bias_r[0:1, :]
  b2 = bias_r[1:2, :]
  bqr = bias_r[2:3, 0:na]
  wa = bias_r[3:4, 0:na]
  wab = bias_r[4, 0]
  pre1 = (jnp.dot(rel.astype(bf16), w1a_r[...], preferred_element_type=f32)
          + tw1 + b1)
  t1 = jnp.where(pre1 >= 0, pre1, 0.01 * pre1)
  pre2 = jnp.dot(t1.astype(bf16), w2_r[...], preferred_element_type=f32) + b2
  h2 = jnp.where(pre2 >= 0, pre2, 0.01 * pre2)
  hr = h2 + rel
  att = (jnp.dot(hs.astype(bf16), ws_r[...], preferred_element_type=f32)
         + jnp.dot(hr.astype(bf16), wr_r[...], preferred_element_type=f32)
         + qa[:, 0:na] + bqr)
  att = jnp.maximum(att, 0.0)
  logit = jnp.sum(att * wa, axis=1, keepdims=True) + wab
  alpha = jax.nn.sigmoid(logit)
  out_r[...] = alpha * (hs + hr)


def _final_block_kernel(agg_r, agg2_r, wh_r, out_r):
  a = (agg_r[0] + agg_r[1]) + (agg2_r[0] + agg2_r[1])
  out_r[...] = jnp.dot(a, wh_r[...], preferred_element_type=jnp.float32)


def kernel(q_sub, q_rel, hidden, edges, n_node, rela_embed, time_embed,
           Ws, Wr, fuse_w1, fuse_b1, fuse_w2, fuse_b2, Wqr_w, Wqr_b,
           wa_w, wa_b, Wh):
  E = edges.shape[0]
  N = hidden.shape[0]
  D = hidden.shape[1]
  A = Ws.shape[0]
  NQ = q_rel.shape[0]

  e0 = edges[:, 0].astype(jnp.int32)
  e2 = edges[:, 2].astype(jnp.int32)
  e4 = edges[:, 4].astype(jnp.int32)
  e5 = edges[:, 5].astype(jnp.int32)
  e6 = edges[:, 6].astype(jnp.int32)

  w1bT = fuse_w1[:, D:].T
  TW = time_embed.shape[0]
  tw1 = pl.pallas_call(
      _matmul_block_kernel,
      grid=(TW // 1000,),
      in_specs=[
          pl.BlockSpec((1000, time_embed.shape[1]), lambda i: (i, 0)),
          pl.BlockSpec((time_embed.shape[1], D), lambda i: (0, 0)),
      ],
      out_specs=pl.BlockSpec((1000, D), lambda i: (i, 0)),
      out_shape=jax.ShapeDtypeStruct((TW, D), jnp.float32),
  )(time_embed, w1bT)

  NR = rela_embed.shape[0]
  wqrT_pad = jnp.zeros((D, D), jnp.float32).at[:, :A].set(Wqr_w.T)
  rq128 = pl.pallas_call(
      _matmul_block_kernel,
      grid=(1,),
      in_specs=[
          pl.BlockSpec((NR, D), lambda i: (0, 0)),
          pl.BlockSpec((D, D), lambda i: (0, 0)),
      ],
      out_specs=pl.BlockSpec((NR, D), lambda i: (0, 0)),
      out_shape=jax.ShapeDtypeStruct((NR, D), jnp.float32),
  )(rela_embed, wqrT_pad)

  blk = NW * CH
  NQP = ((NQ + blk - 1) // blk) * blk
  qrel_pad = jnp.zeros((NQP,), jnp.int32).at[:NQ].set(q_rel.astype(jnp.int32))
  qa_tab = _sc_rows_gather(NQP, D, NQP // blk)(qrel_pad, rq128)

  w1aT = fuse_w1[:, :D].T.astype(jnp.bfloat16)
  w2T = fuse_w2.T.astype(jnp.bfloat16)
  wsT = Ws.T.astype(jnp.bfloat16)
  wrT = Wr.T.astype(jnp.bfloat16)
  bias_pack = jnp.zeros((8, D), jnp.float32)
  bias_pack = bias_pack.at[0, :].set(fuse_b1)
  bias_pack = bias_pack.at[1, :].set(fuse_b2)
  bias_pack = bias_pack.at[2, :A].set(Wqr_b)
  bias_pack = bias_pack.at[3, :A].set(wa_w[0])
  bias_pack = bias_pack.at[4, 0].set(wa_b[0])

  ES = E // NCHUNK
  per_w = ES // NW
  n_ch = per_w // CH
  gather_fn = _sc_edge_gather(per_w, n_ch, D)
  BE = 512

  def edge_tc(rel_g, tw1_g, hs_g, qa_g):
    return pl.pallas_call(
        _edge_block_kernel,
        grid=(ES // BE,),
        in_specs=[
            pl.BlockSpec((BE, D), lambda i: (i, 0)),
            pl.BlockSpec((BE, D), lambda i: (i, 0)),
            pl.BlockSpec((BE, D), lambda i: (i, 0)),
            pl.BlockSpec((BE, D), lambda i: (i, 0)),
            pl.BlockSpec((D, D), lambda i: (0, 0)),
            pl.BlockSpec((D, D), lambda i: (0, 0)),
            pl.BlockSpec((D, A), lambda i: (0, 0)),
            pl.BlockSpec((D, A), lambda i: (0, 0)),
            pl.BlockSpec((8, D), lambda i: (0, 0)),
        ],
        out_specs=pl.BlockSpec((BE, D), lambda i: (i, 0)),
        out_shape=jax.ShapeDtypeStruct((ES, D), jnp.float32),
    )(rel_g, tw1_g, hs_g, qa_g, w1aT, w2T, wsT, wrT, bias_pack)

  msgs = []
  objs = []
  for k in range(NCHUNK):
    sl = slice(k * ES, (k + 1) * ES)
    qa_g, rel_g, hs_g, tw1_g = gather_fn(
        e0[sl], e2[sl], e4[sl], e6[sl], qa_tab, rela_embed, hidden, tw1)
    msgs.append(edge_tc(rel_g, tw1_g, hs_g, qa_g))
    objs.append(e5[sl])

  zeros_nd = jnp.zeros((N, D), jnp.float32)
  agg_a = _sc_scatter(per_w, n_ch, N, D, NCHUNK - 1)(
      *objs[:-1], *msgs[:-1], zeros_nd)
  agg_b = _sc_scatter(per_w, n_ch, N, D, 1)(objs[-1], msgs[-1], zeros_nd)

  RB = 1000
  out = pl.pallas_call(
      _final_block_kernel,
      grid=(N // RB,),
      in_specs=[
          pl.BlockSpec((NC, RB, D), lambda i: (0, i, 0)),
          pl.BlockSpec((NC, RB, D), lambda i: (0, i, 0)),
          pl.BlockSpec((D, D), lambda i: (0, 0)),
      ],
      out_specs=pl.BlockSpec((RB, D), lambda i: (i, 0)),
      out_shape=jax.ShapeDtypeStruct((N, D), jnp.float32),
  )(agg_a, agg_b, Wh.T)
  return out
